# baseline (speedup 1.0000x reference)
import functools

import jax
import jax.numpy as jnp
from jax import lax
from jax.experimental import pallas as pl
from jax.experimental.pallas import tpu as pltpu
from jax.experimental.pallas import tpu_sc as plsc

N_NODES = 10000
N_EDGES = 320000
N_CITIES = 100
N_ETH = 128

NC = 2
NS = 16
L = 16
NW = NC * NS

K = 128
EPW = -(-N_EDGES // NW)
CHUNKS = 81
CP = CHUNKS + 3
NB = CHUNKS // 3
ROWS_PAD = 10112
FULL_CHUNKS = N_NODES // K
TAIL = N_NODES - FULL_CHUNKS * K
TAIL_TILE = FULL_CHUNKS - 4 * NS
P2_T = -(-FULL_CHUNKS // NS)
CB = 8
GRP = K // L


def _sc_body(pop, coef, n2c, src3, dst3, lap3,
             out,
             acc, rows0, rows1, rows2, lapb0, lapb1, lapb2,
             didx0, didx1, didx2, sidx0, sidx1, sidx2,
             sG0, sG1, sG2, sS0, sS1, sS2,
             sDD0, sDD1, sDD2, sDL0, sDL1, sDL2,
             sR0, sR1, sR2):
    c = lax.axis_index("c")
    s = lax.axis_index("s")
    w = c * NS + s

    rows = [rows0, rows1, rows2]
    lapb = [lapb0, lapb1, lapb2]
    didx = [didx0, didx1, didx2]
    sidx = [sidx0, sidx1, sidx2]
    sG = [sG0, sG1, sG2]
    sS = [sS0, sS1, sS2]
    sDD = [sDD0, sDD1, sDD2]
    sDL = [sDL0, sDL1, sDL2]
    sR = [sR0, sR1, sR2]

    zvec = jnp.zeros((L,), jnp.float32)

    def zero_row(r, _):
        for b in range(CB):
            rows0[r, pl.ds(b * L, L)] = zvec
        return 0

    lax.fori_loop(0, K, zero_row, 0)
    for t in range(P2_T):
        chunk = s + NS * t

        @pl.when(chunk < FULL_CHUNKS)
        def _():
            pltpu.sync_copy(rows0, acc.at[pl.ds(chunk * K, K)])

    @pl.when(s == TAIL_TILE)
    def _():
        pltpu.sync_copy(rows0.at[pl.ds(0, TAIL)],
                        acc.at[pl.ds(FULL_CHUNKS * K, TAIL)])
    plsc.subcore_barrier()

    def g_start(q, i):
        pltpu.async_copy(pop.at[didx[q]], rows[q], sG[q])

    def g_wait(q):
        pltpu.make_async_copy(pop.at[didx[q]], rows[q], sG[q]).wait()

    def sc_start(q):
        pltpu.make_async_copy(rows[q], acc.at[sidx[q]], sS[q]).start(add=True)

    def sc_wait(q):
        pltpu.make_async_copy(rows[q], acc.at[sidx[q]], sS[q]).wait()

    def dd_start(q, i):
        pltpu.async_copy(dst3.at[w, i], didx[q], sDD[q])

    def dd_wait(q):
        pltpu.make_async_copy(dst3.at[w, 0], didx[q], sDD[q]).wait()

    def dl_start(q, i):
        pltpu.async_copy(lap3.at[w, i], lapb[q], sDL[q])

    def dl_wait(q):
        pltpu.make_async_copy(lap3.at[w, 0], lapb[q], sDL[q]).wait()

    def sr_start(q, i):
        pltpu.async_copy(src3.at[w, i], sidx[q], sR[q])

    def sr_wait(q):
        pltpu.make_async_copy(src3.at[w, 0], sidx[q], sR[q]).wait()

    for q in range(3):
        pltpu.sync_copy(dst3.at[w, q], didx[q])
        pltpu.sync_copy(lap3.at[w, q], lapb[q])
        sr_start(q, q)
        g_start(q, q)

    def body(p, _):
        for q in range(3):
            i = 3 * p + q
            g_wait(q)
            dd_start(q, i + 3)

            def scale_grp(g, _):
                lv16 = lapb[q][pl.ds(g * L, L)]
                for u in range(L):
                    e = g * L + u
                    lv = lv16[u]
                    for b in range(CB):
                        sl = (e, pl.ds(b * L, L))
                        rows[q][sl] = rows[q][sl] * lv
                return 0

            lax.fori_loop(0, GRP, scale_grp, 0)
            dl_start(q, i + 3)
            sr_wait(q)
            sc_start(q)
            qm = (q + 2) % 3

            def rearm():
                sc_wait(qm)
                dd_wait(qm)
                dl_wait(qm)
                g_start(qm, i + 2)
                sr_start(qm, i + 2)

            if q == 0:
                pl.when(p > 0)(rearm)
            else:
                rearm()
        return 0

    lax.fori_loop(0, NB, body, 0)
    g_wait(0)
    g_wait(1)
    sc_wait(2)
    dd_wait(2)
    dl_wait(2)
    sr_wait(0)
    sr_wait(1)
    plsc.subcore_barrier()

    def mul_body(r4, _):
        for u in range(4):
            r = r4 * 4 + u
            for b in range(CB):
                sl = (r, pl.ds(b * L, L))
                rows0[sl] = rows0[sl] * rows1[sl]
        return 0

    def p2_chunk(t, _):
        chunk = s + NS * t

        @pl.when(chunk < FULL_CHUNKS)
        def _():
            r0 = chunk * K
            pltpu.sync_copy(n2c.at[pl.ds(r0, K)], didx0)
            pltpu.async_copy(coef.at[didx0], rows1, sG0)
            pltpu.sync_copy(acc.at[pl.ds(r0, K)], rows0)
            pltpu.make_async_copy(coef.at[didx0], rows1, sG0).wait()
            lax.fori_loop(0, K // 4, mul_body, 0)
            pltpu.sync_copy(rows0, out.at[pl.ds(c * ROWS_PAD + r0, K)])
        return 0

    lax.fori_loop(0, P2_T, p2_chunk, 0)

    @pl.when(s == TAIL_TILE)
    def _():
        r0 = FULL_CHUNKS * K
        pltpu.sync_copy(n2c.at[pl.ds(r0, K)], didx0)
        pltpu.async_copy(coef.at[didx0.at[pl.ds(0, TAIL)]],
                         rows1.at[pl.ds(0, TAIL)], sG0)
        pltpu.sync_copy(acc.at[pl.ds(r0, TAIL)], rows0.at[pl.ds(0, TAIL)])
        pltpu.make_async_copy(coef.at[didx0.at[pl.ds(0, TAIL)]],
                              rows1.at[pl.ds(0, TAIL)], sG0).wait()
        lax.fori_loop(0, TAIL // 4, mul_body, 0)
        pltpu.sync_copy(rows0.at[pl.ds(0, TAIL)],
                        out.at[pl.ds(c * ROWS_PAD + r0, TAIL)])


def _add_body(a_ref, b_ref, o_ref):
    o_ref[...] = a_ref[...] + b_ref[...]


@jax.jit
def _run(pop, coef, n2c, src3, dst3, lap3):
    f32 = jnp.float32
    i32 = jnp.int32
    kern = pl.kernel(
        _sc_body,
        out_type=jax.ShapeDtypeStruct((NC * ROWS_PAD, N_ETH), f32),
        mesh=plsc.VectorSubcoreMesh(
            core_axis_name="c", subcore_axis_name="s",
            num_cores=NC, num_subcores=NS,
        ),
        scratch_types=[
            pltpu.VMEM_SHARED((N_NODES, N_ETH), f32),
            pltpu.VMEM((K, N_ETH), f32),
            pltpu.VMEM((K, N_ETH), f32),
            pltpu.VMEM((K, N_ETH), f32),
            pltpu.VMEM((K,), f32),
            pltpu.VMEM((K,), f32),
            pltpu.VMEM((K,), f32),
            pltpu.VMEM((K,), i32),
            pltpu.VMEM((K,), i32),
            pltpu.VMEM((K,), i32),
            pltpu.VMEM((K,), i32),
            pltpu.VMEM((K,), i32),
            pltpu.VMEM((K,), i32),
        ] + [pltpu.SemaphoreType.DMA] * 15,
    )
    partial = kern(pop, coef, n2c, src3, dst3, lap3)

    final = pl.pallas_call(
        _add_body,
        out_shape=jax.ShapeDtypeStruct((ROWS_PAD, N_ETH), f32),
        grid=(ROWS_PAD // K,),
        in_specs=[
            pl.BlockSpec((K, N_ETH), lambda i: (i, 0)),
            pl.BlockSpec((K, N_ETH), lambda i: (i + ROWS_PAD // K, 0)),
        ],
        out_specs=pl.BlockSpec((K, N_ETH), lambda i: (i, 0)),
    )(partial, partial)
    return final


def kernel(population, diffusion_coef, lap_values, src, dst, node_to_city):
    n2c = jnp.pad(node_to_city, (0, ROWS_PAD - N_NODES))
    pad_e = NW * EPW - N_EDGES
    src_p = jnp.pad(src, (0, pad_e))
    dst_p = jnp.pad(dst, (0, pad_e))
    lap_p = jnp.pad(lap_values, (0, pad_e))
    src3 = jnp.pad(src_p.reshape(NW, EPW),
                   ((0, 0), (0, CP * K - EPW))).reshape(NW, CP, K)
    dst3 = jnp.pad(dst_p.reshape(NW, EPW),
                   ((0, 0), (0, CP * K - EPW))).reshape(NW, CP, K)
    lap3 = jnp.pad(lap_p.reshape(NW, EPW),
                   ((0, 0), (0, CP * K - EPW))).reshape(NW, CP, K)
    final = _run(population, diffusion_coef, n2c, src3, dst3, lap3)
    return final[:N_NODES]


# speedup vs baseline: 1.7865x; 1.7865x over previous
import functools

import jax
import jax.numpy as jnp
from jax import lax
from jax.experimental import pallas as pl
from jax.experimental.pallas import tpu as pltpu
from jax.experimental.pallas import tpu_sc as plsc

N_NODES = 10000
N_EDGES = 320000
N_CITIES = 100
N_ETH = 128

NC = 2
NS = 16
L = 16
NW = NC * NS

K = 128
EPW = -(-N_EDGES // NW)
CHUNKS = -(-EPW // K)
CP = CHUNKS + 1
ROWS_PAD = 10112
FULL_CHUNKS = N_NODES // K
TAIL = N_NODES - FULL_CHUNKS * K
TAIL_TILE = FULL_CHUNKS - 4 * NS
P2_T = -(-FULL_CHUNKS // NS)
CB = 8
GRP = K // L


def _sc_body(pop, coef, n2c, src3, dst3, lap3,
             out,
             acc, rows0, rows1, lapb0, lapb1,
             didx0, didx1, sidx0, sidx1,
             sG0, sG1, sS0, sS1, sDD, sDL, sSR):
    c = lax.axis_index("c")
    s = lax.axis_index("s")
    w = c * NS + s

    rows = [rows0, rows1]
    lapb = [lapb0, lapb1]
    didx = [didx0, didx1]
    sidx = [sidx0, sidx1]
    sG = [sG0, sG1]
    sS = [sS0, sS1]

    zvec = jnp.zeros((L,), jnp.float32)

    def zero_row(r, _):
        for b in range(CB):
            rows0[r, pl.ds(b * L, L)] = zvec
        return 0

    lax.fori_loop(0, K, zero_row, 0)
    for t in range(P2_T):
        chunk = s + NS * t

        @pl.when(chunk < FULL_CHUNKS)
        def _():
            pltpu.sync_copy(rows0, acc.at[pl.ds(chunk * K, K)])

    @pl.when(s == TAIL_TILE)
    def _():
        pltpu.sync_copy(rows0.at[pl.ds(0, TAIL)],
                        acc.at[pl.ds(FULL_CHUNKS * K, TAIL)])
    plsc.subcore_barrier()

    def chunk_body(i, _):
        pltpu.sync_copy(dst3.at[w, i], didx0)
        pltpu.sync_copy(lap3.at[w, i], lapb0)
        pltpu.sync_copy(src3.at[w, i], sidx0)
        pltpu.sync_copy(pop.at[didx0], rows0)

        def scale_grp(g, _):
            lv16 = lapb0[pl.ds(g * L, L)]
            for u in range(L):
                e = g * L + u
                lv = lv16[u]
                for b in range(CB):
                    sl = (e, pl.ds(b * L, L))
                    rows0[sl] = rows0[sl] * lv
            return 0

        lax.fori_loop(0, GRP, scale_grp, 0)
        pltpu.make_async_copy(rows0, acc.at[sidx0], sS0).start(add=True)
        pltpu.make_async_copy(rows0, acc.at[sidx0], sS0).wait()
        return 0

    lax.fori_loop(0, CHUNKS, chunk_body, 0)
    plsc.subcore_barrier()

    def mul_body(r4, _):
        for u in range(4):
            r = r4 * 4 + u
            for b in range(CB):
                sl = (r, pl.ds(b * L, L))
                rows0[sl] = rows0[sl] * rows1[sl]
        return 0

    def p2_chunk(t, _):
        chunk = s + NS * t

        @pl.when(chunk < FULL_CHUNKS)
        def _():
            r0 = chunk * K
            pltpu.sync_copy(n2c.at[pl.ds(r0, K)], didx0)
            pltpu.async_copy(coef.at[didx0], rows1, sG0)
            pltpu.sync_copy(acc.at[pl.ds(r0, K)], rows0)
            pltpu.make_async_copy(coef.at[didx0], rows1, sG0).wait()
            lax.fori_loop(0, K // 4, mul_body, 0)
            pltpu.sync_copy(rows0, out.at[pl.ds(c * ROWS_PAD + r0, K)])
        return 0

    lax.fori_loop(0, P2_T, p2_chunk, 0)

    @pl.when(s == TAIL_TILE)
    def _():
        r0 = FULL_CHUNKS * K
        pltpu.sync_copy(n2c.at[pl.ds(r0, K)], didx0)
        pltpu.async_copy(coef.at[didx0.at[pl.ds(0, TAIL)]],
                         rows1.at[pl.ds(0, TAIL)], sG0)
        pltpu.sync_copy(acc.at[pl.ds(r0, TAIL)], rows0.at[pl.ds(0, TAIL)])
        pltpu.make_async_copy(coef.at[didx0.at[pl.ds(0, TAIL)]],
                              rows1.at[pl.ds(0, TAIL)], sG0).wait()
        lax.fori_loop(0, TAIL // 4, mul_body, 0)
        pltpu.sync_copy(rows0.at[pl.ds(0, TAIL)],
                        out.at[pl.ds(c * ROWS_PAD + r0, TAIL)])


def _add_body(a_ref, b_ref, o_ref):
    o_ref[...] = a_ref[...] + b_ref[...]


@jax.jit
def _run(pop, coef, n2c, src3, dst3, lap3):
    f32 = jnp.float32
    i32 = jnp.int32
    kern = pl.kernel(
        _sc_body,
        out_type=jax.ShapeDtypeStruct((NC * ROWS_PAD, N_ETH), f32),
        mesh=plsc.VectorSubcoreMesh(
            core_axis_name="c", subcore_axis_name="s",
            num_cores=NC, num_subcores=NS,
        ),
        scratch_types=[
            pltpu.VMEM_SHARED((N_NODES, N_ETH), f32),
            pltpu.VMEM((K, N_ETH), f32),
            pltpu.VMEM((K, N_ETH), f32),
            pltpu.VMEM((K,), f32),
            pltpu.VMEM((K,), f32),
            pltpu.VMEM((K,), i32),
            pltpu.VMEM((K,), i32),
            pltpu.VMEM((K,), i32),
            pltpu.VMEM((K,), i32),
        ] + [pltpu.SemaphoreType.DMA] * 7,
    )
    partial = kern(pop, coef, n2c, src3, dst3, lap3)

    final = pl.pallas_call(
        _add_body,
        out_shape=jax.ShapeDtypeStruct((ROWS_PAD, N_ETH), f32),
        grid=(ROWS_PAD // K,),
        in_specs=[
            pl.BlockSpec((K, N_ETH), lambda i: (i, 0)),
            pl.BlockSpec((K, N_ETH), lambda i: (i + ROWS_PAD // K, 0)),
        ],
        out_specs=pl.BlockSpec((K, N_ETH), lambda i: (i, 0)),
    )(partial, partial)
    return final


def kernel(population, diffusion_coef, lap_values, src, dst, node_to_city):
    n2c = jnp.pad(node_to_city, (0, ROWS_PAD - N_NODES))
    pad_e = NW * EPW - N_EDGES
    src_p = jnp.pad(src, (0, pad_e))
    dst_p = jnp.pad(dst, (0, pad_e))
    lap_p = jnp.pad(lap_values, (0, pad_e))
    src3 = jnp.pad(src_p.reshape(NW, EPW),
                   ((0, 0), (0, CP * K - EPW))).reshape(NW, CP, K)
    dst3 = jnp.pad(dst_p.reshape(NW, EPW),
                   ((0, 0), (0, CP * K - EPW))).reshape(NW, CP, K)
    lap3 = jnp.pad(lap_p.reshape(NW, EPW),
                   ((0, 0), (0, CP * K - EPW))).reshape(NW, CP, K)
    final = _run(population, diffusion_coef, n2c, src3, dst3, lap3)
    return final[:N_NODES]
